# baseline (speedup 1.0000x reference)
import functools

import jax
import jax.numpy as jnp
import numpy as np
from jax import lax
from jax.experimental import pallas as pl
from jax.experimental.pallas import tpu as pltpu
from jax.experimental.pallas import tpu_sc as plsc

_NC = 2
_NS = 16
_NW = _NC * _NS
_LANES = 16

_G = 8

_CHANGE = np.array([
    [3 ** -0.5, 0, 0, 0, 3 ** -0.5, 0, 0, 0, 3 ** -0.5],
    [0, 0, 0, 0, 0, 2 ** -0.5, 0, -(2 ** -0.5), 0],
    [0, 0, -(2 ** -0.5), 0, 0, 0, 2 ** -0.5, 0, 0],
    [0, 2 ** -0.5, 0, -(2 ** -0.5), 0, 0, 0, 0, 0],
    [0, 0, 0.5 ** 0.5, 0, 0, 0, 0.5 ** 0.5, 0, 0],
    [0, 2 ** -0.5, 0, 2 ** -0.5, 0, 0, 0, 0, 0],
    [-(6 ** -0.5), 0, 0, 0, 2 * 6 ** -0.5, 0, 0, 0, -(6 ** -0.5)],
    [0, 0, 0, 0, 0, 2 ** -0.5, 0, 2 ** -0.5, 0],
    [2 ** -0.5, 0, 0, 0, 0, 0, 0, 0, -(2 ** -0.5)],
], dtype=np.float32)
_CHANGE_SUB = _CHANGE[[0, 4, 5, 6, 7, 8], :]

_SPH_C = float(np.sqrt(5.0 / (4.0 * np.pi)))
_SQRT3 = float(np.sqrt(3.0))


def _silu(x):
    return x * (0.5 * jnp.tanh(0.5 * x) + 0.5)


def _sc_hist_body(cpt, dst_hbm, zeros_hbm, out_hbm, dst_v, hist_v):
    wid = lax.axis_index("s") * _NC + lax.axis_index("c")
    base = wid * cpt * _LANES
    pltpu.sync_copy(dst_hbm.at[pl.ds(base, cpt * _LANES)], dst_v)
    pltpu.sync_copy(zeros_hbm, hist_v)
    ones = jnp.ones((_LANES,), jnp.int32)

    def body(i, carry):
        idx = dst_v[pl.ds(i * _LANES, _LANES)]
        plsc.addupdate_scatter(hist_v, [idx], ones)
        return carry

    lax.fori_loop(0, cpt, body, 0)
    pltpu.sync_copy(hist_v, out_hbm.at[wid])


def _sc_hist(dst_pad, np_pad, cpt):
    zeros = jnp.zeros((np_pad,), jnp.int32)
    mesh = plsc.VectorSubcoreMesh(
        core_axis_name="c", subcore_axis_name="s",
        num_cores=_NC, num_subcores=_NS)
    fn = functools.partial(
        pl.kernel,
        out_type=jax.ShapeDtypeStruct((_NW, np_pad), jnp.int32),
        mesh=mesh,
        scratch_types=[
            pltpu.VMEM((cpt * _LANES,), jnp.int32),
            pltpu.VMEM((np_pad,), jnp.int32),
        ],
        compiler_params=pltpu.CompilerParams(needs_layout_passes=False),
    )(functools.partial(_sc_hist_body, cpt))
    return fn(dst_pad, zeros)


def _tc_weights_body(n, hist_ref, batch_ref, out_ref, wsum_ref):
    counts = jnp.sum(hist_ref[...], axis=0, keepdims=True)
    denom = jnp.maximum(counts, 1).astype(jnp.float32)
    out_ref[...] = 1.0 / denom
    np_pad = counts.shape[1]
    col = jax.lax.broadcasted_iota(jnp.int32, (1, np_pad), 1)
    live = jnp.logical_and(counts > 0, col < n)
    b = batch_ref[...]
    cols = []
    for gg in range(_G):
        m = jnp.logical_and(live, b == gg).astype(jnp.float32)
        cols.append(jnp.sum(m, axis=1, keepdims=True))
    wsum_ref[...] = jnp.concatenate(cols, axis=1)


def _tc_weights(hist_part, batch_pad, n):
    np_pad = hist_part.shape[1]
    return pl.pallas_call(
        functools.partial(_tc_weights_body, n),
        out_shape=(
            jax.ShapeDtypeStruct((1, np_pad), jnp.float32),
            jax.ShapeDtypeStruct((1, _G), jnp.float32),
        ),
    )(hist_part, batch_pad.reshape(1, np_pad))


def _sc_gather_body(cpt, np_pad, dst_hbm, w_hbm, b_hbm,
                    w_out, g_out, dst_v, wtab, btab, we_v, ge_v):
    wid = lax.axis_index("s") * _NC + lax.axis_index("c")
    base = wid * cpt * _LANES
    pltpu.sync_copy(dst_hbm.at[pl.ds(base, cpt * _LANES)], dst_v)
    pltpu.sync_copy(w_hbm, wtab)
    pltpu.sync_copy(b_hbm, btab)

    def body(i, carry):
        sl = pl.ds(i * _LANES, _LANES)
        idx = dst_v[sl]
        we_v[sl] = plsc.load_gather(wtab, [idx])
        ge_v[sl] = plsc.load_gather(btab, [idx])
        return carry

    lax.fori_loop(0, cpt, body, 0)
    pltpu.sync_copy(we_v, w_out.at[pl.ds(base, cpt * _LANES)])
    pltpu.sync_copy(ge_v, g_out.at[pl.ds(base, cpt * _LANES)])


def _sc_gather(dst_pad, w_node, batch_pad, cpt):
    e_pad = dst_pad.shape[0]
    np_pad = w_node.shape[0]
    mesh = plsc.VectorSubcoreMesh(
        core_axis_name="c", subcore_axis_name="s",
        num_cores=_NC, num_subcores=_NS)
    fn = functools.partial(
        pl.kernel,
        out_type=(
            jax.ShapeDtypeStruct((e_pad,), jnp.float32),
            jax.ShapeDtypeStruct((e_pad,), jnp.int32),
        ),
        mesh=mesh,
        scratch_types=[
            pltpu.VMEM((cpt * _LANES,), jnp.int32),
            pltpu.VMEM((np_pad,), jnp.float32),
            pltpu.VMEM((np_pad,), jnp.int32),
            pltpu.VMEM((cpt * _LANES,), jnp.float32),
            pltpu.VMEM((cpt * _LANES,), jnp.int32),
        ],
        compiler_params=pltpu.CompilerParams(needs_layout_passes=False),
    )(functools.partial(_sc_gather_body, cpt, np_pad))
    return fn(dst_pad, w_node, batch_pad)


def _tc_main_body(nb, ff_ref, ev_ref, w_ref, g_ref,
                  ws1_ref, bs1_ref, ws2_ref, wi1_ref, bi1_ref, wi2_ref,
                  b2_ref, msub_ref, wsum_ref, out_ref, acc_ref):
    i = pl.program_id(0)
    bf = jnp.bfloat16
    grow = g_ref[...]
    gsel = jax.lax.broadcasted_iota(jnp.int32, (_G, 1), 0)
    r8 = jnp.where(grow == gsel, w_ref[...], 0.0).astype(bf)

    def col(x, w2):
        gk = jax.lax.dot_general(r8, x, (((1,), (0,)), ((), ())),
                                 preferred_element_type=jnp.float32)
        return jnp.dot(gk, w2, preferred_element_type=jnp.float32)

    ff = ff_ref[...].astype(bf)
    a = jnp.dot(ff, ws1_ref[...].astype(bf),
                preferred_element_type=jnp.float32).astype(bf)
    s = _silu(a + bs1_ref[...].astype(bf))
    cols = [col(s, ws2_ref[...])]
    t = jnp.dot(ff, wi1_ref[...].astype(bf),
                preferred_element_type=jnp.float32).astype(bf)
    ev = ev_ref[...]
    x, y, z = ev[0:1, :], ev[1:2, :], ev[2:3, :]
    rn = 1.0 / (jnp.sqrt(x * x + y * y + z * z) + 1e-12)
    nx, ny, nz = x * rn, y * rn, z * rn
    sph5 = jnp.concatenate([
        _SPH_C * _SQRT3 * nx * nz,
        _SPH_C * _SQRT3 * nx * ny,
        _SPH_C * (ny * ny - 0.5 * (nx * nx + nz * nz)),
        _SPH_C * _SQRT3 * ny * nz,
        _SPH_C * (_SQRT3 / 2.0) * (nz * nz - nx * nx),
    ], axis=0).astype(bf)
    eye5 = jnp.eye(5, dtype=bf)
    spht = jax.lax.dot_general(sph5, eye5, (((0,), (0,)), ((), ())),
                               preferred_element_type=jnp.float32
                               ).astype(bf)
    bi1 = bi1_ref[...].astype(bf)
    wi2 = wi2_ref[...]
    for k in range(5):
        p = _silu(spht[:, k:k + 1] * t + bi1)
        cols.append(col(p, wi2))
    bins = jnp.concatenate(cols, axis=1)

    @pl.when(i == 0)
    def _():
        acc_ref[...] = bins

    @pl.when(i > 0)
    def _():
        acc_ref[...] = acc_ref[...] + bins

    @pl.when(i == nb - 1)
    def _():
        b6 = jnp.concatenate(
            [b2_ref[:, 0:1]] + [b2_ref[:, 1:2]] * 5, axis=1)
        acc = acc_ref[...] + wsum_ref[...] * b6
        msub = msub_ref[...]
        out = acc[:, 0:1] * msub[0:1, :]
        for k in range(1, 6):
            out = out + acc[:, k:k + 1] * msub[k:k + 1, :]
        out_ref[...] = out


def _tc_main(ff, ev, w2d, g2d, Ws1, bs1, Ws2, Wi1, bi1, Wi2, b2, wsum, be):
    e, h = ff.shape
    nb = e // be
    msub = jnp.asarray(_CHANGE_SUB)
    grid = (nb,)
    edge_spec = lambda c: pl.BlockSpec((be, c), lambda i: (i, 0))
    const_spec = lambda s: pl.BlockSpec(s, lambda i: (0, 0))
    return pl.pallas_call(
        functools.partial(_tc_main_body, nb),
        grid=grid,
        in_specs=[
            edge_spec(h),
            pl.BlockSpec((3, be), lambda i: (0, i)),
            pl.BlockSpec((1, be), lambda i: (0, i)),
            pl.BlockSpec((1, be), lambda i: (0, i)),
            const_spec((h, h)),
            const_spec((1, h)),
            const_spec((h, 1)),
            const_spec((h, h)),
            const_spec((1, h)),
            const_spec((h, 1)),
            const_spec((1, 2)),
            const_spec((6, 9)),
            const_spec((_G, 1)),
        ],
        out_specs=pl.BlockSpec((_G, 9), lambda i: (0, 0)),
        out_shape=jax.ShapeDtypeStruct((_G, 9), jnp.float32),
        scratch_shapes=[pltpu.VMEM((_G, 6), jnp.float32)],
        compiler_params=pltpu.CompilerParams(
            fuse_transposed_lhs_in_matmul=True),
    )(ff, ev, w2d, g2d, Ws1, bs1, Ws2, Wi1, bi1, Wi2, b2, msub, wsum)


def kernel(force_features, edge_vectors, edge_index_dst, batch_idx,
           Ws1, bs1, Ws2, bs2, Wi1, bi1, Wi2, bi2):
    e, h = force_features.shape
    n = batch_idx.shape[0]

    np_pad = ((n + 1 + 127) // 128) * 128
    chunks = -(-e // _LANES)
    cpt = -(-chunks // _NW)
    e_pad = cpt * _LANES * _NW

    dst = edge_index_dst.astype(jnp.int32)
    dst_pad = jnp.concatenate(
        [dst, jnp.full((e_pad - e,), n, jnp.int32)])
    batch_pad = jnp.concatenate(
        [batch_idx.astype(jnp.int32),
         jnp.zeros((np_pad - n,), jnp.int32)])

    hist_part = _sc_hist(dst_pad, np_pad, cpt)
    w_node2, wsum = _tc_weights(hist_part, batch_pad, n)
    w_node = w_node2.reshape(np_pad)
    w_e, g_e = _sc_gather(dst_pad, w_node, batch_pad, cpt)

    be = 6400
    w2d = w_e.reshape(1, e_pad)
    g2d = g_e.reshape(1, e_pad)
    b2 = jnp.stack([bs2[0], bi2[0]]).reshape(1, 2)
    out = _tc_main(force_features, edge_vectors.T, w2d, g2d,
                   Ws1, bs1.reshape(1, h), Ws2.reshape(h, 1),
                   Wi1, bi1.reshape(1, h), Wi2.reshape(h, 1), b2,
                   wsum.reshape(_G, 1), be)
    return out.reshape(_G, 3, 3)


# speedup vs baseline: 1.0063x; 1.0063x over previous
import functools

import jax
import jax.numpy as jnp
import numpy as np
from jax import lax
from jax.experimental import pallas as pl
from jax.experimental.pallas import tpu as pltpu
from jax.experimental.pallas import tpu_sc as plsc

_NC = 2
_NS = 16
_NW = _NC * _NS
_LANES = 16

_G = 8

_CHANGE = np.array([
    [3 ** -0.5, 0, 0, 0, 3 ** -0.5, 0, 0, 0, 3 ** -0.5],
    [0, 0, 0, 0, 0, 2 ** -0.5, 0, -(2 ** -0.5), 0],
    [0, 0, -(2 ** -0.5), 0, 0, 0, 2 ** -0.5, 0, 0],
    [0, 2 ** -0.5, 0, -(2 ** -0.5), 0, 0, 0, 0, 0],
    [0, 0, 0.5 ** 0.5, 0, 0, 0, 0.5 ** 0.5, 0, 0],
    [0, 2 ** -0.5, 0, 2 ** -0.5, 0, 0, 0, 0, 0],
    [-(6 ** -0.5), 0, 0, 0, 2 * 6 ** -0.5, 0, 0, 0, -(6 ** -0.5)],
    [0, 0, 0, 0, 0, 2 ** -0.5, 0, 2 ** -0.5, 0],
    [2 ** -0.5, 0, 0, 0, 0, 0, 0, 0, -(2 ** -0.5)],
], dtype=np.float32)
_CHANGE_SUB = _CHANGE[[0, 4, 5, 6, 7, 8], :]

_SPH_C = float(np.sqrt(5.0 / (4.0 * np.pi)))
_SQRT3 = float(np.sqrt(3.0))


def _silu(x):
    return x * (0.5 * jnp.tanh(0.5 * x) + 0.5)


def _sc_hist_body(cpt, dst_hbm, zeros_hbm, out_hbm, dst_v, hist_v):
    wid = lax.axis_index("s") * _NC + lax.axis_index("c")
    base = wid * cpt * _LANES
    pltpu.sync_copy(dst_hbm.at[pl.ds(base, cpt * _LANES)], dst_v)
    pltpu.sync_copy(zeros_hbm, hist_v)
    ones = jnp.ones((_LANES,), jnp.int32)

    def body(i, carry):
        idx = dst_v[pl.ds(i * _LANES, _LANES)]
        plsc.addupdate_scatter(hist_v, [idx], ones)
        return carry

    lax.fori_loop(0, cpt, body, 0)
    pltpu.sync_copy(hist_v, out_hbm.at[wid])


def _sc_hist(dst_pad, np_pad, cpt):
    zeros = jnp.zeros((np_pad,), jnp.int32)
    mesh = plsc.VectorSubcoreMesh(
        core_axis_name="c", subcore_axis_name="s",
        num_cores=_NC, num_subcores=_NS)
    fn = functools.partial(
        pl.kernel,
        out_type=jax.ShapeDtypeStruct((_NW, np_pad), jnp.int32),
        mesh=mesh,
        scratch_types=[
            pltpu.VMEM((cpt * _LANES,), jnp.int32),
            pltpu.VMEM((np_pad,), jnp.int32),
        ],
        compiler_params=pltpu.CompilerParams(needs_layout_passes=False),
    )(functools.partial(_sc_hist_body, cpt))
    return fn(dst_pad, zeros)


def _tc_weights_body(hist_ref, out_ref):
    counts = jnp.sum(hist_ref[...], axis=0, keepdims=True)
    denom = jnp.maximum(counts, 1).astype(jnp.float32)
    out_ref[...] = 1.0 / denom


def _tc_weights(hist_part):
    np_pad = hist_part.shape[1]
    return pl.pallas_call(
        _tc_weights_body,
        out_shape=jax.ShapeDtypeStruct((1, np_pad), jnp.float32),
    )(hist_part)


def _sc_gather_body(cpt, np_pad, dst_hbm, w_hbm, b_hbm,
                    w_out, g_out, dst_v, wtab, btab, we_v, ge_v):
    wid = lax.axis_index("s") * _NC + lax.axis_index("c")
    base = wid * cpt * _LANES
    pltpu.sync_copy(dst_hbm.at[pl.ds(base, cpt * _LANES)], dst_v)
    pltpu.sync_copy(w_hbm, wtab)
    pltpu.sync_copy(b_hbm, btab)

    def body(i, carry):
        sl = pl.ds(i * _LANES, _LANES)
        idx = dst_v[sl]
        we_v[sl] = plsc.load_gather(wtab, [idx])
        ge_v[sl] = plsc.load_gather(btab, [idx])
        return carry

    lax.fori_loop(0, cpt, body, 0)
    pltpu.sync_copy(we_v, w_out.at[pl.ds(base, cpt * _LANES)])
    pltpu.sync_copy(ge_v, g_out.at[pl.ds(base, cpt * _LANES)])


def _sc_gather(dst_pad, w_node, batch_pad, cpt):
    e_pad = dst_pad.shape[0]
    np_pad = w_node.shape[0]
    mesh = plsc.VectorSubcoreMesh(
        core_axis_name="c", subcore_axis_name="s",
        num_cores=_NC, num_subcores=_NS)
    fn = functools.partial(
        pl.kernel,
        out_type=(
            jax.ShapeDtypeStruct((e_pad,), jnp.float32),
            jax.ShapeDtypeStruct((e_pad,), jnp.int32),
        ),
        mesh=mesh,
        scratch_types=[
            pltpu.VMEM((cpt * _LANES,), jnp.int32),
            pltpu.VMEM((np_pad,), jnp.float32),
            pltpu.VMEM((np_pad,), jnp.int32),
            pltpu.VMEM((cpt * _LANES,), jnp.float32),
            pltpu.VMEM((cpt * _LANES,), jnp.int32),
        ],
        compiler_params=pltpu.CompilerParams(needs_layout_passes=False),
    )(functools.partial(_sc_gather_body, cpt, np_pad))
    return fn(dst_pad, w_node, batch_pad)


def _tc_main_body(nb, ff_ref, ev_ref, w_ref, g_ref,
                  ws1_ref, bs1_ref, ws2_ref, wi1_ref, bi1_ref, wi2_ref,
                  b2_ref, msub_ref, out_ref, acc_ref):
    i = pl.program_id(0)
    bf = jnp.bfloat16
    grow = g_ref[...]
    gsel = jax.lax.broadcasted_iota(jnp.int32, (_G, 1), 0)
    r8 = jnp.where(grow == gsel, w_ref[...], 0.0).astype(bf)

    def col(x, w2):
        gk = jax.lax.dot_general(r8, x, (((1,), (0,)), ((), ())),
                                 preferred_element_type=jnp.float32)
        return jnp.dot(gk, w2, preferred_element_type=jnp.float32)

    ff = ff_ref[...].astype(bf)
    a = jnp.dot(ff, ws1_ref[...].astype(bf),
                preferred_element_type=jnp.float32).astype(bf)
    s = _silu(a + bs1_ref[...].astype(bf))
    cols = [col(s, ws2_ref[...])]
    t = jnp.dot(ff, wi1_ref[...].astype(bf),
                preferred_element_type=jnp.float32).astype(bf)
    ev = ev_ref[...]
    x, y, z = ev[0:1, :], ev[1:2, :], ev[2:3, :]
    rn = 1.0 / (jnp.sqrt(x * x + y * y + z * z) + 1e-12)
    nx, ny, nz = x * rn, y * rn, z * rn
    sph5 = jnp.concatenate([
        _SPH_C * _SQRT3 * nx * nz,
        _SPH_C * _SQRT3 * nx * ny,
        _SPH_C * (ny * ny - 0.5 * (nx * nx + nz * nz)),
        _SPH_C * _SQRT3 * ny * nz,
        _SPH_C * (_SQRT3 / 2.0) * (nz * nz - nx * nx),
    ], axis=0).astype(bf)
    eye5 = jnp.eye(5, dtype=bf)
    spht = jax.lax.dot_general(sph5, eye5, (((0,), (0,)), ((), ())),
                               preferred_element_type=jnp.float32
                               ).astype(bf)
    bi1 = bi1_ref[...].astype(bf)
    wi2 = wi2_ref[...]
    for k in range(5):
        p = _silu(spht[:, k:k + 1] * t + bi1)
        cols.append(col(p, wi2))
    ones_col = jnp.ones((r8.shape[1], 1), bf)
    cols.append(jnp.dot(r8, ones_col, preferred_element_type=jnp.float32))
    bins = jnp.concatenate(cols, axis=1)

    @pl.when(i == 0)
    def _():
        acc_ref[...] = bins

    @pl.when(i > 0)
    def _():
        acc_ref[...] = acc_ref[...] + bins

    @pl.when(i == nb - 1)
    def _():
        b6 = jnp.concatenate(
            [b2_ref[:, 0:1]] + [b2_ref[:, 1:2]] * 5, axis=1)
        accw = acc_ref[...]
        acc = accw[:, 0:6] + accw[:, 6:7] * b6
        msub = msub_ref[...]
        out = acc[:, 0:1] * msub[0:1, :]
        for k in range(1, 6):
            out = out + acc[:, k:k + 1] * msub[k:k + 1, :]
        out_ref[...] = out


def _tc_main(ff, ev, w2d, g2d, Ws1, bs1, Ws2, Wi1, bi1, Wi2, b2, be):
    e, h = ff.shape
    nb = e // be
    msub = jnp.asarray(_CHANGE_SUB)
    grid = (nb,)
    edge_spec = lambda c: pl.BlockSpec((be, c), lambda i: (i, 0))
    const_spec = lambda s: pl.BlockSpec(s, lambda i: (0, 0))
    return pl.pallas_call(
        functools.partial(_tc_main_body, nb),
        grid=grid,
        in_specs=[
            edge_spec(h),
            pl.BlockSpec((3, be), lambda i: (0, i)),
            pl.BlockSpec((1, be), lambda i: (0, i)),
            pl.BlockSpec((1, be), lambda i: (0, i)),
            const_spec((h, h)),
            const_spec((1, h)),
            const_spec((h, 1)),
            const_spec((h, h)),
            const_spec((1, h)),
            const_spec((h, 1)),
            const_spec((1, 2)),
            const_spec((6, 9)),
        ],
        out_specs=pl.BlockSpec((_G, 9), lambda i: (0, 0)),
        out_shape=jax.ShapeDtypeStruct((_G, 9), jnp.float32),
        scratch_shapes=[pltpu.VMEM((_G, 7), jnp.float32)],
        compiler_params=pltpu.CompilerParams(
            fuse_transposed_lhs_in_matmul=True),
    )(ff, ev, w2d, g2d, Ws1, bs1, Ws2, Wi1, bi1, Wi2, b2, msub)


def kernel(force_features, edge_vectors, edge_index_dst, batch_idx,
           Ws1, bs1, Ws2, bs2, Wi1, bi1, Wi2, bi2):
    e, h = force_features.shape
    n = batch_idx.shape[0]

    np_pad = ((n + 1 + 127) // 128) * 128
    chunks = -(-e // _LANES)
    cpt = -(-chunks // _NW)
    e_pad = cpt * _LANES * _NW

    dst = edge_index_dst.astype(jnp.int32)
    dst_pad = jnp.concatenate(
        [dst, jnp.full((e_pad - e,), n, jnp.int32)])
    batch_pad = jnp.concatenate(
        [batch_idx.astype(jnp.int32),
         jnp.zeros((np_pad - n,), jnp.int32)])

    hist_part = _sc_hist(dst_pad, np_pad, cpt)
    w_node = _tc_weights(hist_part).reshape(np_pad)
    w_e, g_e = _sc_gather(dst_pad, w_node, batch_pad, cpt)

    be = 16000
    w2d = w_e.reshape(1, e_pad)
    g2d = g_e.reshape(1, e_pad)
    b2 = jnp.stack([bs2[0], bi2[0]]).reshape(1, 2)
    out = _tc_main(force_features, edge_vectors.T, w2d, g2d,
                   Ws1, bs1.reshape(1, h), Ws2.reshape(h, 1),
                   Wi1, bi1.reshape(1, h), Wi2.reshape(h, 1), b2, be)
    return out.reshape(_G, 3, 3)


# speedup vs baseline: 1.0475x; 1.0410x over previous
import functools

import jax
import jax.numpy as jnp
import numpy as np
from jax import lax
from jax.experimental import pallas as pl
from jax.experimental.pallas import tpu as pltpu
from jax.experimental.pallas import tpu_sc as plsc

_NC = 2
_NS = 16
_NW = _NC * _NS
_LANES = 16

_G = 8

_CHANGE = np.array([
    [3 ** -0.5, 0, 0, 0, 3 ** -0.5, 0, 0, 0, 3 ** -0.5],
    [0, 0, 0, 0, 0, 2 ** -0.5, 0, -(2 ** -0.5), 0],
    [0, 0, -(2 ** -0.5), 0, 0, 0, 2 ** -0.5, 0, 0],
    [0, 2 ** -0.5, 0, -(2 ** -0.5), 0, 0, 0, 0, 0],
    [0, 0, 0.5 ** 0.5, 0, 0, 0, 0.5 ** 0.5, 0, 0],
    [0, 2 ** -0.5, 0, 2 ** -0.5, 0, 0, 0, 0, 0],
    [-(6 ** -0.5), 0, 0, 0, 2 * 6 ** -0.5, 0, 0, 0, -(6 ** -0.5)],
    [0, 0, 0, 0, 0, 2 ** -0.5, 0, 2 ** -0.5, 0],
    [2 ** -0.5, 0, 0, 0, 0, 0, 0, 0, -(2 ** -0.5)],
], dtype=np.float32)
_CHANGE_SUB = _CHANGE[[0, 4, 5, 6, 7, 8], :]

_SPH_C = float(np.sqrt(5.0 / (4.0 * np.pi)))
_SQRT3 = float(np.sqrt(3.0))


def _silu(x):
    return x * (0.5 * jnp.tanh(0.5 * x) + 0.5)


def _sc_hist_body(cpt, dst_hbm, zeros_hbm, out_hbm, dst_v, hist_v):
    wid = lax.axis_index("s") * _NC + lax.axis_index("c")
    base = wid * cpt * _LANES
    pltpu.sync_copy(dst_hbm.at[pl.ds(base, cpt * _LANES)], dst_v)
    pltpu.sync_copy(zeros_hbm, hist_v)
    ones = jnp.ones((_LANES,), jnp.int32)

    def body(i, carry):
        idx = dst_v[pl.ds(i * _LANES, _LANES)]
        plsc.addupdate_scatter(hist_v, [idx], ones)
        return carry

    lax.fori_loop(0, cpt, body, 0)
    pltpu.sync_copy(hist_v, out_hbm.at[wid])


def _sc_hist(dst_pad, np_pad, cpt):
    zeros = jnp.zeros((np_pad,), jnp.int32)
    mesh = plsc.VectorSubcoreMesh(
        core_axis_name="c", subcore_axis_name="s",
        num_cores=_NC, num_subcores=_NS)
    fn = functools.partial(
        pl.kernel,
        out_type=jax.ShapeDtypeStruct((_NW, np_pad), jnp.int32),
        mesh=mesh,
        scratch_types=[
            pltpu.VMEM((cpt * _LANES,), jnp.int32),
            pltpu.VMEM((np_pad,), jnp.int32),
        ],
        compiler_params=pltpu.CompilerParams(needs_layout_passes=False),
    )(functools.partial(_sc_hist_body, cpt))
    return fn(dst_pad, zeros)


def _tc_weights_body(n, hist_ref, batch_ref, out_ref, wsum_ref):
    counts = jnp.sum(hist_ref[...], axis=0, keepdims=True)
    denom = jnp.maximum(counts, 1).astype(jnp.float32)
    out_ref[...] = 1.0 / denom
    np_pad = counts.shape[1]
    col = jax.lax.broadcasted_iota(jnp.int32, (1, np_pad), 1)
    live = jnp.logical_and(counts > 0, col < n)
    b = batch_ref[...]
    cols = []
    for gg in range(_G):
        m = jnp.logical_and(live, b == gg).astype(jnp.float32)
        cols.append(jnp.sum(m, axis=1, keepdims=True))
    wsum_ref[...] = jnp.concatenate(cols, axis=1)


def _tc_weights(hist_part, batch_pad, n):
    np_pad = hist_part.shape[1]
    return pl.pallas_call(
        functools.partial(_tc_weights_body, n),
        out_shape=(
            jax.ShapeDtypeStruct((1, np_pad), jnp.float32),
            jax.ShapeDtypeStruct((1, _G), jnp.float32),
        ),
    )(hist_part, batch_pad.reshape(1, np_pad))


def _sc_gather_body(cpt, np_pad, dst_hbm, w_hbm, b_hbm,
                    w_out, g_out, dst_v, wtab, btab, we_v, ge_v):
    wid = lax.axis_index("s") * _NC + lax.axis_index("c")
    base = wid * cpt * _LANES
    pltpu.sync_copy(dst_hbm.at[pl.ds(base, cpt * _LANES)], dst_v)
    pltpu.sync_copy(w_hbm, wtab)
    pltpu.sync_copy(b_hbm, btab)

    @plsc.parallel_loop(0, cpt * _LANES, step=_LANES, unroll=8)
    def body(i):
        sl = pl.ds(i, _LANES)
        idx = dst_v[sl]
        we_v[sl] = plsc.load_gather(wtab, [idx])
        ge_v[sl] = plsc.load_gather(btab, [idx])

    pltpu.sync_copy(we_v, w_out.at[pl.ds(base, cpt * _LANES)])
    pltpu.sync_copy(ge_v, g_out.at[pl.ds(base, cpt * _LANES)])


def _sc_gather(dst_pad, w_node, batch_pad, cpt):
    e_pad = dst_pad.shape[0]
    np_pad = w_node.shape[0]
    mesh = plsc.VectorSubcoreMesh(
        core_axis_name="c", subcore_axis_name="s",
        num_cores=_NC, num_subcores=_NS)
    fn = functools.partial(
        pl.kernel,
        out_type=(
            jax.ShapeDtypeStruct((e_pad,), jnp.float32),
            jax.ShapeDtypeStruct((e_pad,), jnp.int32),
        ),
        mesh=mesh,
        scratch_types=[
            pltpu.VMEM((cpt * _LANES,), jnp.int32),
            pltpu.VMEM((np_pad,), jnp.float32),
            pltpu.VMEM((np_pad,), jnp.int32),
            pltpu.VMEM((cpt * _LANES,), jnp.float32),
            pltpu.VMEM((cpt * _LANES,), jnp.int32),
        ],
        compiler_params=pltpu.CompilerParams(needs_layout_passes=False),
    )(functools.partial(_sc_gather_body, cpt, np_pad))
    return fn(dst_pad, w_node, batch_pad)


def _tc_main_body(nb, ff_ref, ev_ref, w_ref, g_ref,
                  ws1_ref, bs1_ref, ws2_ref, wi1_ref, bi1_ref, wi2_ref,
                  b2_ref, msub_ref, wsum_ref, out_ref, acc_ref):
    i = pl.program_id(0)
    bf = jnp.bfloat16
    grow = g_ref[...]
    gsel = jax.lax.broadcasted_iota(jnp.int32, (_G, 1), 0)
    r8 = jnp.where(grow == gsel, w_ref[...], 0.0).astype(bf)

    def col(x, w2):
        gk = jax.lax.dot_general(r8, x, (((1,), (0,)), ((), ())),
                                 preferred_element_type=jnp.float32)
        return jnp.dot(gk, w2, preferred_element_type=jnp.float32)

    ff = ff_ref[...].astype(bf)
    a = jnp.dot(ff, ws1_ref[...].astype(bf),
                preferred_element_type=jnp.float32).astype(bf)
    s = _silu(a + bs1_ref[...].astype(bf))
    cols = [col(s, ws2_ref[...])]
    t = jnp.dot(ff, wi1_ref[...].astype(bf),
                preferred_element_type=jnp.float32).astype(bf)
    ev = ev_ref[...]
    x, y, z = ev[0:1, :], ev[1:2, :], ev[2:3, :]
    rn = 1.0 / (jnp.sqrt(x * x + y * y + z * z) + 1e-12)
    nx, ny, nz = x * rn, y * rn, z * rn
    sph5 = jnp.concatenate([
        _SPH_C * _SQRT3 * nx * nz,
        _SPH_C * _SQRT3 * nx * ny,
        _SPH_C * (ny * ny - 0.5 * (nx * nx + nz * nz)),
        _SPH_C * _SQRT3 * ny * nz,
        _SPH_C * (_SQRT3 / 2.0) * (nz * nz - nx * nx),
    ], axis=0).astype(bf)
    eye5 = jnp.eye(5, dtype=bf)
    spht = jax.lax.dot_general(sph5, eye5, (((0,), (0,)), ((), ())),
                               preferred_element_type=jnp.float32
                               ).astype(bf)
    bi1 = bi1_ref[...].astype(bf)
    wi2 = wi2_ref[...]
    for k in range(5):
        p = _silu(spht[:, k:k + 1] * t + bi1)
        cols.append(col(p, wi2))
    bins = jnp.concatenate(cols, axis=1)

    @pl.when(i == 0)
    def _():
        acc_ref[...] = bins

    @pl.when(i > 0)
    def _():
        acc_ref[...] = acc_ref[...] + bins

    @pl.when(i == nb - 1)
    def _():
        b6 = jnp.concatenate(
            [b2_ref[:, 0:1]] + [b2_ref[:, 1:2]] * 5, axis=1)
        acc = acc_ref[...] + wsum_ref[...] * b6
        msub = msub_ref[...]
        out = acc[:, 0:1] * msub[0:1, :]
        for k in range(1, 6):
            out = out + acc[:, k:k + 1] * msub[k:k + 1, :]
        out_ref[...] = out


def _tc_main(ff, ev, w2d, g2d, Ws1, bs1, Ws2, Wi1, bi1, Wi2, b2, wsum, be):
    e, h = ff.shape
    nb = e // be
    msub = jnp.asarray(_CHANGE_SUB)
    grid = (nb,)
    edge_spec = lambda c: pl.BlockSpec((be, c), lambda i: (i, 0))
    const_spec = lambda s: pl.BlockSpec(s, lambda i: (0, 0))
    return pl.pallas_call(
        functools.partial(_tc_main_body, nb),
        grid=grid,
        in_specs=[
            edge_spec(h),
            pl.BlockSpec((3, be), lambda i: (0, i)),
            pl.BlockSpec((1, be), lambda i: (0, i)),
            pl.BlockSpec((1, be), lambda i: (0, i)),
            const_spec((h, h)),
            const_spec((1, h)),
            const_spec((h, 1)),
            const_spec((h, h)),
            const_spec((1, h)),
            const_spec((h, 1)),
            const_spec((1, 2)),
            const_spec((6, 9)),
            const_spec((_G, 1)),
        ],
        out_specs=pl.BlockSpec((_G, 9), lambda i: (0, 0)),
        out_shape=jax.ShapeDtypeStruct((_G, 9), jnp.float32),
        scratch_shapes=[pltpu.VMEM((_G, 6), jnp.float32)],
        compiler_params=pltpu.CompilerParams(
            fuse_transposed_lhs_in_matmul=True),
    )(ff, ev, w2d, g2d, Ws1, bs1, Ws2, Wi1, bi1, Wi2, b2, msub, wsum)


def kernel(force_features, edge_vectors, edge_index_dst, batch_idx,
           Ws1, bs1, Ws2, bs2, Wi1, bi1, Wi2, bi2):
    e, h = force_features.shape
    n = batch_idx.shape[0]

    np_pad = ((n + 1 + 127) // 128) * 128
    chunks = -(-e // _LANES)
    cpt = -(-chunks // _NW)
    e_pad = cpt * _LANES * _NW

    dst = edge_index_dst.astype(jnp.int32)
    dst_pad = jnp.concatenate(
        [dst, jnp.full((e_pad - e,), n, jnp.int32)])
    batch_pad = jnp.concatenate(
        [batch_idx.astype(jnp.int32),
         jnp.zeros((np_pad - n,), jnp.int32)])

    hist_part = _sc_hist(dst_pad, np_pad, cpt)
    w_node2, wsum = _tc_weights(hist_part, batch_pad, n)
    w_node = w_node2.reshape(np_pad)
    w_e, g_e = _sc_gather(dst_pad, w_node, batch_pad, cpt)

    be = 16000
    w2d = w_e.reshape(1, e_pad)
    g2d = g_e.reshape(1, e_pad)
    b2 = jnp.stack([bs2[0], bi2[0]]).reshape(1, 2)
    out = _tc_main(force_features, edge_vectors.T, w2d, g2d,
                   Ws1, bs1.reshape(1, h), Ws2.reshape(h, 1),
                   Wi1, bi1.reshape(1, h), Wi2.reshape(h, 1), b2,
                   wsum.reshape(_G, 1), be)
    return out.reshape(_G, 3, 3)


# speedup vs baseline: 1.0489x; 1.0013x over previous
import functools

import jax
import jax.numpy as jnp
import numpy as np
from jax import lax
from jax.experimental import pallas as pl
from jax.experimental.pallas import tpu as pltpu
from jax.experimental.pallas import tpu_sc as plsc

_NC = 2
_NS = 16
_NW = _NC * _NS
_LANES = 16

_G = 8

_CHANGE = np.array([
    [3 ** -0.5, 0, 0, 0, 3 ** -0.5, 0, 0, 0, 3 ** -0.5],
    [0, 0, 0, 0, 0, 2 ** -0.5, 0, -(2 ** -0.5), 0],
    [0, 0, -(2 ** -0.5), 0, 0, 0, 2 ** -0.5, 0, 0],
    [0, 2 ** -0.5, 0, -(2 ** -0.5), 0, 0, 0, 0, 0],
    [0, 0, 0.5 ** 0.5, 0, 0, 0, 0.5 ** 0.5, 0, 0],
    [0, 2 ** -0.5, 0, 2 ** -0.5, 0, 0, 0, 0, 0],
    [-(6 ** -0.5), 0, 0, 0, 2 * 6 ** -0.5, 0, 0, 0, -(6 ** -0.5)],
    [0, 0, 0, 0, 0, 2 ** -0.5, 0, 2 ** -0.5, 0],
    [2 ** -0.5, 0, 0, 0, 0, 0, 0, 0, -(2 ** -0.5)],
], dtype=np.float32)
_CHANGE_SUB = _CHANGE[[0, 4, 5, 6, 7, 8], :]

_SPH_C = float(np.sqrt(5.0 / (4.0 * np.pi)))
_SQRT3 = float(np.sqrt(3.0))


def _silu_half(h):
    return h * jnp.tanh(h) + h


def _sc_hist_body(cpt, dst_hbm, zeros_hbm, out_hbm, dst_v, hist_v):
    wid = lax.axis_index("s") * _NC + lax.axis_index("c")
    base = wid * cpt * _LANES
    pltpu.sync_copy(dst_hbm.at[pl.ds(base, cpt * _LANES)], dst_v)
    pltpu.sync_copy(zeros_hbm, hist_v)
    ones = jnp.ones((_LANES,), jnp.int32)

    def body(i, carry):
        idx = dst_v[pl.ds(i * _LANES, _LANES)]
        plsc.addupdate_scatter(hist_v, [idx], ones)
        return carry

    lax.fori_loop(0, cpt, body, 0)
    pltpu.sync_copy(hist_v, out_hbm.at[wid])


def _sc_hist(dst_pad, np_pad, cpt):
    zeros = jnp.zeros((np_pad,), jnp.int32)
    mesh = plsc.VectorSubcoreMesh(
        core_axis_name="c", subcore_axis_name="s",
        num_cores=_NC, num_subcores=_NS)
    fn = functools.partial(
        pl.kernel,
        out_type=jax.ShapeDtypeStruct((_NW, np_pad), jnp.int32),
        mesh=mesh,
        scratch_types=[
            pltpu.VMEM((cpt * _LANES,), jnp.int32),
            pltpu.VMEM((np_pad,), jnp.int32),
        ],
        compiler_params=pltpu.CompilerParams(needs_layout_passes=False),
    )(functools.partial(_sc_hist_body, cpt))
    return fn(dst_pad, zeros)


def _tc_weights_body(n, hist_ref, batch_ref, out_ref, wsum_ref):
    counts = jnp.sum(hist_ref[...], axis=0, keepdims=True)
    denom = jnp.maximum(counts, 1).astype(jnp.float32)
    out_ref[...] = 1.0 / denom
    np_pad = counts.shape[1]
    col = jax.lax.broadcasted_iota(jnp.int32, (1, np_pad), 1)
    live = jnp.logical_and(counts > 0, col < n)
    b = batch_ref[...]
    cols = []
    for gg in range(_G):
        m = jnp.logical_and(live, b == gg).astype(jnp.float32)
        cols.append(jnp.sum(m, axis=1, keepdims=True))
    wsum_ref[...] = jnp.concatenate(cols, axis=1)


def _tc_weights(hist_part, batch_pad, n):
    np_pad = hist_part.shape[1]
    return pl.pallas_call(
        functools.partial(_tc_weights_body, n),
        out_shape=(
            jax.ShapeDtypeStruct((1, np_pad), jnp.float32),
            jax.ShapeDtypeStruct((1, _G), jnp.float32),
        ),
    )(hist_part, batch_pad.reshape(1, np_pad))


def _sc_gather_body(cpt, np_pad, dst_hbm, w_hbm, b_hbm,
                    w_out, g_out, dst_v, wtab, btab, we_v, ge_v):
    wid = lax.axis_index("s") * _NC + lax.axis_index("c")
    base = wid * cpt * _LANES
    pltpu.sync_copy(dst_hbm.at[pl.ds(base, cpt * _LANES)], dst_v)
    pltpu.sync_copy(w_hbm, wtab)
    pltpu.sync_copy(b_hbm, btab)

    @plsc.parallel_loop(0, cpt * _LANES, step=_LANES, unroll=8)
    def body(i):
        sl = pl.ds(i, _LANES)
        idx = dst_v[sl]
        we_v[sl] = plsc.load_gather(wtab, [idx])
        ge_v[sl] = plsc.load_gather(btab, [idx])

    pltpu.sync_copy(we_v, w_out.at[pl.ds(base, cpt * _LANES)])
    pltpu.sync_copy(ge_v, g_out.at[pl.ds(base, cpt * _LANES)])


def _sc_gather(dst_pad, w_node, batch_pad, cpt):
    e_pad = dst_pad.shape[0]
    np_pad = w_node.shape[0]
    mesh = plsc.VectorSubcoreMesh(
        core_axis_name="c", subcore_axis_name="s",
        num_cores=_NC, num_subcores=_NS)
    fn = functools.partial(
        pl.kernel,
        out_type=(
            jax.ShapeDtypeStruct((e_pad,), jnp.float32),
            jax.ShapeDtypeStruct((e_pad,), jnp.int32),
        ),
        mesh=mesh,
        scratch_types=[
            pltpu.VMEM((cpt * _LANES,), jnp.int32),
            pltpu.VMEM((np_pad,), jnp.float32),
            pltpu.VMEM((np_pad,), jnp.int32),
            pltpu.VMEM((cpt * _LANES,), jnp.float32),
            pltpu.VMEM((cpt * _LANES,), jnp.int32),
        ],
        compiler_params=pltpu.CompilerParams(needs_layout_passes=False),
    )(functools.partial(_sc_gather_body, cpt, np_pad))
    return fn(dst_pad, w_node, batch_pad)


def _tc_main_body(nb, ff_ref, ev_ref, w_ref, g_ref,
                  ws1_ref, bs1_ref, ws2_ref, wi1_ref, bi1_ref, wi2_ref,
                  b2_ref, msub_ref, wsum_ref, out_ref, acc_ref):
    i = pl.program_id(0)
    bf = jnp.bfloat16
    grow = g_ref[...]
    gsel = jax.lax.broadcasted_iota(jnp.int32, (_G, 1), 0)
    r8 = jnp.where(grow == gsel, w_ref[...], 0.0).astype(bf)

    def col(x, w2):
        gk = jax.lax.dot_general(r8, x, (((1,), (0,)), ((), ())),
                                 preferred_element_type=jnp.float32)
        return jnp.dot(gk, w2, preferred_element_type=jnp.float32)

    ff = ff_ref[...].astype(bf)
    a = jnp.dot(ff, ws1_ref[...].astype(bf),
                preferred_element_type=jnp.float32).astype(bf)
    s = _silu_half(a + bs1_ref[...].astype(bf))
    cols = [col(s, ws2_ref[...])]
    t = jnp.dot(ff, wi1_ref[...].astype(bf),
                preferred_element_type=jnp.float32).astype(bf)
    ev = ev_ref[...]
    x, y, z = ev[0:1, :], ev[1:2, :], ev[2:3, :]
    rn = 1.0 / (jnp.sqrt(x * x + y * y + z * z) + 1e-12)
    nx, ny, nz = x * rn, y * rn, z * rn
    sph5 = jnp.concatenate([
        _SPH_C * _SQRT3 * nx * nz,
        _SPH_C * _SQRT3 * nx * ny,
        _SPH_C * (ny * ny - 0.5 * (nx * nx + nz * nz)),
        _SPH_C * _SQRT3 * ny * nz,
        _SPH_C * (_SQRT3 / 2.0) * (nz * nz - nx * nx),
    ], axis=0).astype(bf)
    eye5 = jnp.eye(5, dtype=bf)
    spht = jax.lax.dot_general(sph5, eye5, (((0,), (0,)), ((), ())),
                               preferred_element_type=jnp.float32
                               ).astype(bf)
    bi1 = bi1_ref[...].astype(bf)
    wi2 = wi2_ref[...]
    for k in range(5):
        p = _silu_half(spht[:, k:k + 1] * t + bi1)
        cols.append(col(p, wi2))
    bins = jnp.concatenate(cols, axis=1)

    @pl.when(i == 0)
    def _():
        acc_ref[...] = bins

    @pl.when(i > 0)
    def _():
        acc_ref[...] = acc_ref[...] + bins

    @pl.when(i == nb - 1)
    def _():
        b6 = jnp.concatenate(
            [b2_ref[:, 0:1]] + [b2_ref[:, 1:2]] * 5, axis=1)
        acc = acc_ref[...] + wsum_ref[...] * b6
        msub = msub_ref[...]
        out = acc[:, 0:1] * msub[0:1, :]
        for k in range(1, 6):
            out = out + acc[:, k:k + 1] * msub[k:k + 1, :]
        out_ref[...] = out


def _tc_main(ff, ev, w2d, g2d, Ws1, bs1, Ws2, Wi1, bi1, Wi2, b2, wsum, be):
    e, h = ff.shape
    nb = e // be
    msub = jnp.asarray(_CHANGE_SUB)
    grid = (nb,)
    edge_spec = lambda c: pl.BlockSpec((be, c), lambda i: (i, 0))
    const_spec = lambda s: pl.BlockSpec(s, lambda i: (0, 0))
    return pl.pallas_call(
        functools.partial(_tc_main_body, nb),
        grid=grid,
        in_specs=[
            edge_spec(h),
            pl.BlockSpec((3, be), lambda i: (0, i)),
            pl.BlockSpec((1, be), lambda i: (0, i)),
            pl.BlockSpec((1, be), lambda i: (0, i)),
            const_spec((h, h)),
            const_spec((1, h)),
            const_spec((h, 1)),
            const_spec((h, h)),
            const_spec((1, h)),
            const_spec((h, 1)),
            const_spec((1, 2)),
            const_spec((6, 9)),
            const_spec((_G, 1)),
        ],
        out_specs=pl.BlockSpec((_G, 9), lambda i: (0, 0)),
        out_shape=jax.ShapeDtypeStruct((_G, 9), jnp.float32),
        scratch_shapes=[pltpu.VMEM((_G, 6), jnp.float32)],
        compiler_params=pltpu.CompilerParams(
            fuse_transposed_lhs_in_matmul=True),
    )(ff, ev, w2d, g2d, Ws1, bs1, Ws2, Wi1, bi1, Wi2, b2, msub, wsum)


def kernel(force_features, edge_vectors, edge_index_dst, batch_idx,
           Ws1, bs1, Ws2, bs2, Wi1, bi1, Wi2, bi2):
    e, h = force_features.shape
    n = batch_idx.shape[0]

    np_pad = ((n + 1 + 127) // 128) * 128
    chunks = -(-e // _LANES)
    cpt = -(-chunks // _NW)
    e_pad = cpt * _LANES * _NW

    dst = edge_index_dst.astype(jnp.int32)
    dst_pad = jnp.concatenate(
        [dst, jnp.full((e_pad - e,), n, jnp.int32)])
    batch_pad = jnp.concatenate(
        [batch_idx.astype(jnp.int32),
         jnp.zeros((np_pad - n,), jnp.int32)])

    hist_part = _sc_hist(dst_pad, np_pad, cpt)
    w_node2, wsum = _tc_weights(hist_part, batch_pad, n)
    w_node = w_node2.reshape(np_pad)
    w_e, g_e = _sc_gather(dst_pad, w_node, batch_pad, cpt)

    be = 16000
    w2d = w_e.reshape(1, e_pad)
    g2d = g_e.reshape(1, e_pad)
    b2 = jnp.stack([bs2[0], bi2[0]]).reshape(1, 2)
    out = _tc_main(force_features, edge_vectors.T, w2d, g2d,
                   0.5 * Ws1, 0.5 * bs1.reshape(1, h), Ws2.reshape(h, 1),
                   0.5 * Wi1, 0.5 * bi1.reshape(1, h), Wi2.reshape(h, 1),
                   b2, wsum.reshape(_G, 1), be)
    return out.reshape(_G, 3, 3)
